# Initial kernel scaffold; baseline (speedup 1.0000x reference)
import jax
import jax.numpy as jnp
from jax.experimental import pallas as pl


def kernel(inputs, targets):
    raise NotImplementedError("write your pallas kernel here")


# speedup vs baseline: 224.1201x; 224.1201x over previous
import jax
import jax.numpy as jnp
from jax.experimental import pallas as pl
from jax.experimental.pallas import tpu as pltpu

C = 19
B = 8
H = 512
W = 512
BH = 128
NH = H // BH


def _body(x_ref, t_ref, out_ref, d_acc, c_acc):
    b = pl.program_id(0)
    h = pl.program_id(1)

    @pl.when((b == 0) & (h == 0))
    def _init():
        for bb in range(B):
            for cc in range(C):
                d_acc[bb, cc] = 0.0
                c_acc[bb, cc] = 0.0

    x = x_ref[0]
    t = t_ref[0]

    m = jnp.max(x, axis=0)
    s = jnp.zeros_like(m)
    for cc in range(C):
        s = s + jnp.exp(x[cc] - m)
    lse = m + jnp.log(s)

    for cc in range(C):
        mf = (t == cc).astype(jnp.float32)
        d_acc[b, cc] += jnp.sum(mf * (x[cc] - lse))
        c_acc[b, cc] += jnp.sum(mf)

    @pl.when((b == B - 1) & (h == NH - 1))
    def _final():
        total = 0.0
        bins = []
        for cc in range(C):
            g = 0.0
            for bb in range(B):
                g += c_acc[bb, cc]
            bins.append(g)
            total += g
        loss = 0.0
        for bb in range(B):
            num = 0.0
            den = 0.0
            for cc in range(C):
                nz = (bins[cc] != 0.0).astype(jnp.float32)
                wgt = nz * (1.0 - bins[cc] / total) + 1.0
                num += wgt * d_acc[bb, cc]
                den += wgt * c_acc[bb, cc]
            loss += -num / den
        out_ref[0, 0] = loss


def kernel(inputs, targets):
    out = pl.pallas_call(
        _body,
        grid=(B, NH),
        in_specs=[
            pl.BlockSpec((1, C, BH, W), lambda b, h: (b, 0, h, 0)),
            pl.BlockSpec((1, BH, W), lambda b, h: (b, h, 0)),
        ],
        out_specs=pl.BlockSpec(
            (1, 1), lambda b, h: (0, 0), memory_space=pltpu.SMEM
        ),
        out_shape=jax.ShapeDtypeStruct((1, 1), jnp.float32),
        scratch_shapes=[
            pltpu.SMEM((B, C), jnp.float32),
            pltpu.SMEM((B, C), jnp.float32),
        ],
    )(inputs, targets)
    return out.reshape(())
